# Initial kernel scaffold; baseline (speedup 1.0000x reference)
import jax
import jax.numpy as jnp
from jax.experimental import pallas as pl


def kernel(x, edge_index, edge_attr, batch, nn1_w, nn1_b, root1, bias1, bn1_g, bn1_b, pool1_w, nn2_w, nn2_b, root2, bias2, bn2_g, bn2_b, pool2_w, nn3_w, nn3_b, root3, bias3, bn3_g, bn3_b, pool3_w, lin1_w, lin1_b, lin2_w, lin2_b):
    raise NotImplementedError("write your pallas kernel here")


# speedup vs baseline: 2.1649x; 2.1649x over previous
import functools
import math

import jax
import jax.numpy as jnp
from jax import lax
from jax.experimental import pallas as pl
from jax.experimental.pallas import tpu as pltpu
from jax.experimental.pallas import tpu_sc as plsc

N = 10000
E = 160000
D_NODE = 64
D_EDGE = 16
H = 32
EPS = 1e-5

NP_ = 10240
KP = D_EDGE + 1
PW = KP * H
PWP = 640
EAW = 32

NUM_CORES = 2
NUM_SUBCORES = 16
NW = NUM_CORES * NUM_SUBCORES
EPW = E // NW
C = 40
NCHUNK = EPW // C
ROWS_PER_TILE = NP_ // NUM_SUBCORES
ZROWS = 160

_F32_ONE_BITS = 1065353216
_LO0 = -1065353217
_HI0 = 1065353217
_DEAD = -1073741824


def _sc_edge_body(p_hbm, ea_hbm, src_hbm, dst_hbm, out_hbm,
                  idx_s, idx_d, rows, eabuf, msg, zbuf, aggr_sh, sem):
    c = lax.axis_index("c")
    s = lax.axis_index("s")
    wid = s * NUM_CORES + c

    @pl.loop(0, ZROWS)
    def _zero(i):
        z = jnp.zeros((16,), jnp.float32)
        zbuf[i, pl.ds(0, 16)] = z
        zbuf[i, pl.ds(16, 16)] = z

    for j in range(ROWS_PER_TILE // ZROWS):
        pltpu.sync_copy(
            zbuf, aggr_sh.at[pl.ds(s * ROWS_PER_TILE + j * ZROWS, ZROWS)])
    plsc.subcore_barrier()

    base0 = wid * EPW

    @pl.loop(0, NCHUNK)
    def _chunk(j):
        base = base0 + j * C
        pltpu.sync_copy(src_hbm.at[pl.ds(base, C)], idx_s)
        pltpu.sync_copy(dst_hbm.at[pl.ds(base, C)], idx_d)
        pltpu.sync_copy(ea_hbm.at[pl.ds(base, C)], eabuf)
        pltpu.async_copy(p_hbm.at[idx_s], rows, sem).wait()

        @pl.loop(0, C)
        def _edge(e):
            ea0 = eabuf[e, pl.ds(0, 16)]
            ea1 = eabuf[e, pl.ds(16, 16)]
            acc0 = jnp.zeros((16,), jnp.float32)
            acc1 = jnp.zeros((16,), jnp.float32)
            for k in range(KP):
                coef = jnp.full((16,), ea0[k] if k < 16 else ea1[0],
                                jnp.float32)
                r0 = rows[e, pl.ds(k * H, 16)]
                r1 = rows[e, pl.ds(k * H + 16, 16)]
                acc0 = acc0 + coef * r0
                acc1 = acc1 + coef * r1
            msg[e, pl.ds(0, 16)] = acc0
            msg[e, pl.ds(16, 16)] = acc1

        pltpu.sync_copy(msg, aggr_sh.at[idx_d], add=True)

    plsc.subcore_barrier()
    pltpu.sync_copy(
        aggr_sh.at[pl.ds(s * ROWS_PER_TILE, ROWS_PER_TILE)],
        out_hbm.at[c, pl.ds(s * ROWS_PER_TILE, ROWS_PER_TILE)])


def _sc_edge(p, ea_flat, src, dst):
    mesh = plsc.VectorSubcoreMesh(core_axis_name="c", subcore_axis_name="s")
    fn = pl.kernel(
        _sc_edge_body,
        out_type=jax.ShapeDtypeStruct((NUM_CORES, NP_, H), jnp.float32),
        compiler_params=pltpu.CompilerParams(use_tc_tiling_on_sc=False),
        mesh=mesh,
        scratch_types=[
            pltpu.VMEM((C,), jnp.int32),
            pltpu.VMEM((C,), jnp.int32),
            pltpu.VMEM((C, PWP), jnp.float32),
            pltpu.VMEM((C, EAW), jnp.float32),
            pltpu.VMEM((C, H), jnp.float32),
            pltpu.VMEM((ZROWS, H), jnp.float32),
            pltpu.VMEM_SHARED((NP_, H), jnp.float32),
            pltpu.SemaphoreType.DMA,
        ],
    )
    return fn(p, ea_flat, src, dst)


def _mm_block(x_ref, w_ref, o_ref):
    o_ref[...] = jnp.dot(x_ref[...], w_ref[...],
                         preferred_element_type=jnp.float32)


def _p_matmul(h, wfull):
    din = h.shape[1]
    bm = 1024
    return pl.pallas_call(
        _mm_block,
        grid=(NP_ // bm,),
        in_specs=[
            pl.BlockSpec((bm, din), lambda i: (i, 0)),
            pl.BlockSpec((din, PWP), lambda i: (0, 0)),
        ],
        out_specs=pl.BlockSpec((bm, PWP), lambda i: (i, 0)),
        out_shape=jax.ShapeDtypeStruct((NP_, PWP), jnp.float32),
    )(h, wfull)


def _count_ge(skey, t):
    return jnp.sum((skey >= t).astype(jnp.int32))


def _epilogue_body(k, final, aggr_ref, h_ref, m_ref, root_ref, bias_ref,
                   g_ref, b_ref, pw_ref, l1w_ref, l1b_ref, l2w_ref, l2b_ref,
                   *out_refs):
    aggr = aggr_ref[0] + aggr_ref[1]
    m = m_ref[...]
    out = (aggr + jnp.dot(h_ref[...], root_ref[...],
                          preferred_element_type=jnp.float32)
           + bias_ref[...][None, :]) * m
    cnt = jnp.sum(m)
    mean = jnp.sum(out, axis=0, keepdims=True) / cnt
    var = jnp.sum(((out - mean) ** 2) * m, axis=0, keepdims=True) / cnt
    hn = (out - mean) / jnp.sqrt(var + EPS)
    h = jnp.maximum(hn * g_ref[...][None, :] + b_ref[...][None, :], 0.0) * m

    w = pw_ref[...]
    wn = jnp.sqrt(jnp.sum(w * w))
    score = jnp.tanh(jnp.dot(h, (w / wn)[:, None],
                             preferred_element_type=jnp.float32))

    bits = lax.bitcast_convert_type(score, jnp.int32)
    skey = bits ^ jnp.where(bits < 0, jnp.int32(0x7FFFFFFF), jnp.int32(0))
    skey = jnp.where(m > 0, skey, jnp.int32(_DEAD))

    def t_body(_, lohi):
        lo, hi = lohi
        mid = lo + (hi - lo) // 2
        ok = _count_ge(skey, mid) >= k
        return (jnp.where(ok, mid, lo), jnp.where(ok, hi, mid))

    t, _ = lax.fori_loop(0, 32, t_body,
                         (jnp.int32(_LO0), jnp.int32(_HI0)))

    n_gt = jnp.sum((skey > t).astype(jnp.int32))
    r = k - n_gt
    eq = skey == t
    idx = lax.broadcasted_iota(jnp.int32, (NP_, 1), 0)

    def i_body(_, lohi):
        lo, hi = lohi
        mid = (lo + hi) // 2
        ok = jnp.sum((eq & (idx < mid)).astype(jnp.int32)) >= r
        return (jnp.where(ok, lo, mid + 1), jnp.where(ok, mid, hi))

    ilo, _ = lax.fori_loop(0, 15, i_body, (jnp.int32(0), jnp.int32(NP_)))

    m_new = ((skey > t) | (eq & (idx < ilo))).astype(jnp.float32)
    h_out = h * score * m_new

    if final:
        pooled = jnp.sum(h_out, axis=0, keepdims=True) / jnp.sum(m_new)
        z = jnp.maximum(jnp.dot(pooled, l1w_ref[...],
                                preferred_element_type=jnp.float32)
                        + l1b_ref[...][None, :], 0.0)
        o = jax.nn.sigmoid(jnp.dot(z, l2w_ref[...],
                                   preferred_element_type=jnp.float32)
                           + l2b_ref[...][None, :])
        out_refs[0][...] = o
    else:
        out_refs[0][...] = h_out
        out_refs[1][...] = m_new


def _epilogue(k, final, aggr, h, m, root, bias, g, b, pw, l1w, l1b, l2w, l2b):
    if final:
        out_shape = jax.ShapeDtypeStruct((1, 1), jnp.float32)
    else:
        out_shape = (jax.ShapeDtypeStruct((NP_, H), jnp.float32),
                     jax.ShapeDtypeStruct((NP_, 1), jnp.float32))
    return pl.pallas_call(
        functools.partial(_epilogue_body, k, final),
        out_shape=out_shape,
    )(aggr, h, m, root, bias, g, b, pw, l1w, l1b, l2w, l2b)


def _make_wfull(nn_w, nn_b, din):
    w = nn_w.reshape(D_EDGE, din, H)
    b = nn_b.reshape(1, din, H)
    w = jnp.concatenate([w, b], axis=0).transpose(1, 0, 2).reshape(din, PW)
    w = jnp.pad(w, ((0, 0), (0, PWP - PW)))
    return w


def kernel(x, edge_index, edge_attr, batch, nn1_w, nn1_b, root1, bias1,
           bn1_g, bn1_b, pool1_w, nn2_w, nn2_b, root2, bias2, bn2_g, bn2_b,
           pool2_w, nn3_w, nn3_b, root3, bias3, bn3_g, bn3_b, pool3_w,
           lin1_w, lin1_b, lin2_w, lin2_b):
    src = edge_index[0]
    dst = edge_index[1]
    ea_flat = jnp.concatenate(
        [edge_attr, jnp.ones((E, 1), jnp.float32),
         jnp.zeros((E, EAW - KP), jnp.float32)], axis=1)

    x_pad = jnp.pad(x, ((0, NP_ - N), (0, 0)))
    row_idx = jnp.arange(NP_, dtype=jnp.int32)[:, None]
    m0 = (row_idx < N).astype(jnp.float32)

    k1 = math.ceil(0.5 * N)
    k2 = math.ceil(0.5 * k1)
    k3 = math.ceil(0.5 * k2)

    w1 = _make_wfull(nn1_w, nn1_b, D_NODE)
    w2 = _make_wfull(nn2_w, nn2_b, H)
    w3 = _make_wfull(nn3_w, nn3_b, H)

    p1 = _p_matmul(x_pad, w1)
    a1 = _sc_edge(p1, ea_flat, src, dst)
    h1, m1 = _epilogue(k1, False, a1, x_pad, m0, root1, bias1, bn1_g, bn1_b,
                       pool1_w, lin1_w, lin1_b, lin2_w, lin2_b)

    p2 = _p_matmul(h1, w2)
    a2 = _sc_edge(p2, ea_flat, src, dst)
    h2, m2 = _epilogue(k2, False, a2, h1, m1, root2, bias2, bn2_g, bn2_b,
                       pool2_w, lin1_w, lin1_b, lin2_w, lin2_b)

    p3 = _p_matmul(h2, w3)
    a3 = _sc_edge(p3, ea_flat, src, dst)
    out = _epilogue(k3, True, a3, h2, m2, root3, bias3, bn3_g, bn3_b,
                    pool3_w, lin1_w, lin1_b, lin2_w, lin2_b)
    return out


# speedup vs baseline: 3.5671x; 1.6476x over previous
import functools
import math

import jax
import jax.numpy as jnp
from jax import lax
from jax.experimental import pallas as pl
from jax.experimental.pallas import tpu as pltpu
from jax.experimental.pallas import tpu_sc as plsc

N = 10000
E = 160000
D_NODE = 64
D_EDGE = 16
H = 32
EPS = 1e-5

NP_ = 10240
KP = D_EDGE + 1
PW = KP * H
PWP = PW
EAW = 32

NUM_CORES = 2
NUM_SUBCORES = 16
NW = NUM_CORES * NUM_SUBCORES
C = 40
NCHUNK = 126
EPAD = NW * NCHUNK * C
ROWS_PER_TILE = NP_ // NUM_SUBCORES
ZROWS = 160

_F32_ONE_BITS = 1065353216
_LO0 = -1065353217
_HI0 = 1065353217
_DEAD = -1073741824


def _sc_edge_body(p_hbm, ea_hbm, src_hbm, dst_hbm, out_hbm,
                  idxs, idxd, rows, eabuf, msg, zbuf, aggr_sh,
                  semr0, semr1, seme0, seme1):
    c = lax.axis_index("c")
    s = lax.axis_index("s")
    wid = s * NUM_CORES + c
    semr = (semr0, semr1)
    seme = (seme0, seme1)

    @pl.loop(0, ZROWS)
    def _zero(i):
        z = jnp.zeros((16,), jnp.float32)
        zbuf[i, pl.ds(0, 16)] = z
        zbuf[i, pl.ds(16, 16)] = z

    for j in range(ROWS_PER_TILE // ZROWS):
        pltpu.sync_copy(
            zbuf, aggr_sh.at[pl.ds(s * ROWS_PER_TILE + j * ZROWS, ZROWS)])
    plsc.subcore_barrier()

    pltpu.sync_copy(src_hbm.at[wid], idxs)
    pltpu.sync_copy(dst_hbm.at[wid], idxd)

    for b in range(2):
        pltpu.async_copy(p_hbm.at[idxs.at[b]], rows.at[b], semr[b])
        pltpu.async_copy(ea_hbm.at[wid, b], eabuf.at[b], seme[b])

    @pl.loop(0, NCHUNK, step=2)
    def _chunk(j):
        for b in range(2):
            jj = j + b
            pltpu.make_async_copy(
                p_hbm.at[idxs.at[b]], rows.at[b], semr[b]).wait()
            pltpu.make_async_copy(
                ea_hbm.at[wid, 0], eabuf.at[b], seme[b]).wait()

            @pl.loop(0, C)
            def _edge(e):
                ea0 = eabuf[b, e, pl.ds(0, 16)]
                ea1 = eabuf[b, e, pl.ds(16, 16)]
                acc0 = jnp.zeros((16,), jnp.float32)
                acc1 = jnp.zeros((16,), jnp.float32)
                for k in range(KP):
                    coef = jnp.full((16,), ea0[k] if k < 16 else ea1[0],
                                    jnp.float32)
                    r0 = rows[b, e, pl.ds(k * H, 16)]
                    r1 = rows[b, e, pl.ds(k * H + 16, 16)]
                    acc0 = acc0 + coef * r0
                    acc1 = acc1 + coef * r1
                msg[e, pl.ds(0, 16)] = acc0
                msg[e, pl.ds(16, 16)] = acc1

            pltpu.sync_copy(msg, aggr_sh.at[idxd.at[jj]], add=True)

            @pl.when(jj + 2 < NCHUNK)
            def _prefetch():
                pltpu.async_copy(
                    p_hbm.at[idxs.at[jj + 2]], rows.at[b], semr[b])
                pltpu.async_copy(ea_hbm.at[wid, jj + 2], eabuf.at[b], seme[b])

    plsc.subcore_barrier()
    pltpu.sync_copy(
        aggr_sh.at[pl.ds(s * ROWS_PER_TILE, ROWS_PER_TILE)],
        out_hbm.at[c, pl.ds(s * ROWS_PER_TILE, ROWS_PER_TILE)])


def _sc_edge(p, ea_flat, src, dst):
    mesh = plsc.VectorSubcoreMesh(core_axis_name="c", subcore_axis_name="s")
    fn = pl.kernel(
        _sc_edge_body,
        out_type=jax.ShapeDtypeStruct((NUM_CORES, NP_, H), jnp.float32),
        compiler_params=pltpu.CompilerParams(use_tc_tiling_on_sc=False),
        mesh=mesh,
        scratch_types=[
            pltpu.VMEM((NCHUNK, C), jnp.int32),
            pltpu.VMEM((NCHUNK, C), jnp.int32),
            pltpu.VMEM((2, C, PWP), jnp.float32),
            pltpu.VMEM((2, C, EAW), jnp.float32),
            pltpu.VMEM((C, H), jnp.float32),
            pltpu.VMEM((ZROWS, H), jnp.float32),
            pltpu.VMEM_SHARED((NP_, H), jnp.float32),
            pltpu.SemaphoreType.DMA,
            pltpu.SemaphoreType.DMA,
            pltpu.SemaphoreType.DMA,
            pltpu.SemaphoreType.DMA,
        ],
    )
    return fn(p, ea_flat, src, dst)


def _mm_block(x_ref, w_ref, o_ref):
    o_ref[...] = jnp.dot(x_ref[...], w_ref[...],
                         preferred_element_type=jnp.float32)


def _p_matmul(h, wfull):
    din = h.shape[1]
    bm = 1024
    return pl.pallas_call(
        _mm_block,
        grid=(NP_ // bm,),
        in_specs=[
            pl.BlockSpec((bm, din), lambda i: (i, 0)),
            pl.BlockSpec((din, PWP), lambda i: (0, 0)),
        ],
        out_specs=pl.BlockSpec((bm, PWP), lambda i: (i, 0)),
        out_shape=jax.ShapeDtypeStruct((NP_, PWP), jnp.float32),
    )(h, wfull)


def _count_ge(skey, t):
    return jnp.sum((skey >= t).astype(jnp.int32))


def _epilogue_body(k, final, aggr_ref, h_ref, m_ref, root_ref, bias_ref,
                   g_ref, b_ref, pw_ref, l1w_ref, l1b_ref, l2w_ref, l2b_ref,
                   *out_refs):
    aggr = aggr_ref[0] + aggr_ref[1]
    m = m_ref[...]
    out = (aggr + jnp.dot(h_ref[...], root_ref[...],
                          preferred_element_type=jnp.float32)
           + bias_ref[...][None, :]) * m
    cnt = jnp.sum(m)
    mean = jnp.sum(out, axis=0, keepdims=True) / cnt
    var = jnp.sum(((out - mean) ** 2) * m, axis=0, keepdims=True) / cnt
    hn = (out - mean) / jnp.sqrt(var + EPS)
    h = jnp.maximum(hn * g_ref[...][None, :] + b_ref[...][None, :], 0.0) * m

    w = pw_ref[...]
    wn = jnp.sqrt(jnp.sum(w * w))
    score = jnp.tanh(jnp.dot(h, (w / wn)[:, None],
                             preferred_element_type=jnp.float32))

    bits = lax.bitcast_convert_type(score, jnp.int32)
    skey = bits ^ jnp.where(bits < 0, jnp.int32(0x7FFFFFFF), jnp.int32(0))
    skey = jnp.where(m > 0, skey, jnp.int32(_DEAD))

    def t_body(_, lohi):
        lo, hi = lohi
        mid = lo + (hi - lo) // 2
        ok = _count_ge(skey, mid) >= k
        return (jnp.where(ok, mid, lo), jnp.where(ok, hi, mid))

    t, _ = lax.fori_loop(0, 32, t_body,
                         (jnp.int32(_LO0), jnp.int32(_HI0)))

    n_gt = jnp.sum((skey > t).astype(jnp.int32))
    r = k - n_gt
    eq = skey == t
    idx = lax.broadcasted_iota(jnp.int32, (NP_, 1), 0)

    def i_body(_, lohi):
        lo, hi = lohi
        mid = (lo + hi) // 2
        ok = jnp.sum((eq & (idx < mid)).astype(jnp.int32)) >= r
        return (jnp.where(ok, lo, mid + 1), jnp.where(ok, mid, hi))

    ilo, _ = lax.fori_loop(0, 15, i_body, (jnp.int32(0), jnp.int32(NP_)))

    m_new = ((skey > t) | (eq & (idx < ilo))).astype(jnp.float32)
    h_out = h * score * m_new

    if final:
        pooled = jnp.sum(h_out, axis=0, keepdims=True) / jnp.sum(m_new)
        z = jnp.maximum(jnp.dot(pooled, l1w_ref[...],
                                preferred_element_type=jnp.float32)
                        + l1b_ref[...][None, :], 0.0)
        o = jax.nn.sigmoid(jnp.dot(z, l2w_ref[...],
                                   preferred_element_type=jnp.float32)
                           + l2b_ref[...][None, :])
        out_refs[0][...] = o
    else:
        out_refs[0][...] = h_out
        out_refs[1][...] = m_new


def _epilogue(k, final, aggr, h, m, root, bias, g, b, pw, l1w, l1b, l2w, l2b):
    if final:
        out_shape = jax.ShapeDtypeStruct((1, 1), jnp.float32)
    else:
        out_shape = (jax.ShapeDtypeStruct((NP_, H), jnp.float32),
                     jax.ShapeDtypeStruct((NP_, 1), jnp.float32))
    return pl.pallas_call(
        functools.partial(_epilogue_body, k, final),
        out_shape=out_shape,
    )(aggr, h, m, root, bias, g, b, pw, l1w, l1b, l2w, l2b)


def _make_wfull(nn_w, nn_b, din):
    w = nn_w.reshape(D_EDGE, din, H)
    b = nn_b.reshape(1, din, H)
    w = jnp.concatenate([w, b], axis=0).transpose(1, 0, 2).reshape(din, PW)
    w = jnp.pad(w, ((0, 0), (0, PWP - PW)))
    return w


def kernel(x, edge_index, edge_attr, batch, nn1_w, nn1_b, root1, bias1,
           bn1_g, bn1_b, pool1_w, nn2_w, nn2_b, root2, bias2, bn2_g, bn2_b,
           pool2_w, nn3_w, nn3_b, root3, bias3, bn3_g, bn3_b, pool3_w,
           lin1_w, lin1_b, lin2_w, lin2_b):
    src = jnp.pad(edge_index[0], (0, EPAD - E)).reshape(NW, NCHUNK, C)
    dst = jnp.pad(edge_index[1], (0, EPAD - E)).reshape(NW, NCHUNK, C)
    ea_flat = jnp.concatenate(
        [edge_attr, jnp.ones((E, 1), jnp.float32),
         jnp.zeros((E, EAW - KP), jnp.float32)], axis=1)
    ea_flat = jnp.pad(ea_flat, ((0, EPAD - E), (0, 0)))
    ea_flat = ea_flat.reshape(NW, NCHUNK, C, EAW)

    x_pad = jnp.pad(x, ((0, NP_ - N), (0, 0)))
    row_idx = jnp.arange(NP_, dtype=jnp.int32)[:, None]
    m0 = (row_idx < N).astype(jnp.float32)

    k1 = math.ceil(0.5 * N)
    k2 = math.ceil(0.5 * k1)
    k3 = math.ceil(0.5 * k2)

    w1 = _make_wfull(nn1_w, nn1_b, D_NODE)
    w2 = _make_wfull(nn2_w, nn2_b, H)
    w3 = _make_wfull(nn3_w, nn3_b, H)

    p1 = _p_matmul(x_pad, w1)
    a1 = _sc_edge(p1, ea_flat, src, dst)
    h1, m1 = _epilogue(k1, False, a1, x_pad, m0, root1, bias1, bn1_g, bn1_b,
                       pool1_w, lin1_w, lin1_b, lin2_w, lin2_b)

    p2 = _p_matmul(h1, w2)
    a2 = _sc_edge(p2, ea_flat, src, dst)
    h2, m2 = _epilogue(k2, False, a2, h1, m1, root2, bias2, bn2_g, bn2_b,
                       pool2_w, lin1_w, lin1_b, lin2_w, lin2_b)

    p3 = _p_matmul(h2, w3)
    a3 = _sc_edge(p3, ea_flat, src, dst)
    out = _epilogue(k3, True, a3, h2, m2, root3, bias3, bn3_g, bn3_b,
                    pool3_w, lin1_w, lin1_b, lin2_w, lin2_b)
    return out
